# baseline (speedup 1.0000x reference)
import numpy as np
import jax
import jax.numpy as jnp
from jax.experimental import pallas as pl
from jax.experimental.pallas import tpu as pltpu

_GAMMA = 0.5
_UBLK = 40


def _calc_delta_kernel(d2_ref, qd_ref, x_ref, lm_ref, e_ref, t_ref, out_ref):
    w = e_ref.shape[1]
    ub = e_ref.shape[0]
    bb, n = d2_ref.shape
    nchunks = n // ub

    d2 = d2_ref[...]
    rowmin = jnp.min(d2, axis=1, keepdims=True)
    iota = jax.lax.broadcasted_iota(jnp.int32, (bb, n), 1)
    idx = jnp.min(jnp.where(d2 == rowmin, iota, n), axis=1)
    onehot = (iota == idx[:, None]).astype(jnp.float32)
    g = jax.lax.dot_general(
        qd_ref[...], onehot,
        dimension_numbers=(((0,), (1,)), ((), ())),
        preferred_element_type=jnp.float32,
    )
    ht = jnp.exp(-_GAMMA * g)

    xt = jax.lax.dot_general(
        x_ref[...], t_ref[...],
        dimension_numbers=(((1,), (0,)), ((), ())),
        preferred_element_type=jnp.float32,
    )
    for k in range(nchunks):
        h_rep = jax.lax.dot_general(
            ht[k * ub:(k + 1) * ub, :], e_ref[...],
            dimension_numbers=(((0,), (0,)), ((), ())),
            preferred_element_type=jnp.float32,
        )
        out_ref[:, k * w:(k + 1) * w] = h_rep * (xt - lm_ref[0, k * w:(k + 1) * w][None, :])


@jax.jit
def kernel(x, d2, qd, landmarks):
    b, f = x.shape
    n = qd.shape[0]
    ub = _UBLK
    w = ub * f

    lanes = np.arange(w)
    e_mat = jnp.asarray((lanes[None, :] // f) == np.arange(ub)[:, None],
                        dtype=jnp.float32)
    t_mat = jnp.asarray((lanes[None, :] % f) == np.arange(f)[:, None],
                        dtype=jnp.float32)
    lm_flat = landmarks.reshape(1, n * f)

    bb = 128
    out_flat = pl.pallas_call(
        _calc_delta_kernel,
        grid=(b // bb,),
        in_specs=[
            pl.BlockSpec((bb, n), lambda i: (i, 0)),
            pl.BlockSpec((n, n), lambda i: (0, 0)),
            pl.BlockSpec((bb, f), lambda i: (i, 0)),
            pl.BlockSpec((1, n * f), lambda i: (0, 0)),
            pl.BlockSpec((ub, w), lambda i: (0, 0)),
            pl.BlockSpec((f, w), lambda i: (0, 0)),
        ],
        out_specs=pl.BlockSpec((bb, n * f), lambda i: (i, 0)),
        out_shape=jax.ShapeDtypeStruct((b, n * f), jnp.float32),
        compiler_params=pltpu.CompilerParams(
            dimension_semantics=("parallel",),
        ),
    )(d2, qd, x, lm_flat, e_mat, t_mat)

    return out_flat.reshape(b, n, f)
